# baseline (speedup 1.0000x reference)
import functools

import jax
import jax.numpy as jnp
from jax import lax
from jax.experimental import pallas as pl
from jax.experimental.pallas import tpu as pltpu
from jax.experimental.pallas import tpu_sc as plsc

NC, NS = 2, 16
NW = NC * NS


def kernel(sequence, table):
    B, S = sequence.shape
    V, D = table.shape
    BW = B // NW
    assert BW * NW == B and S % 2 == 0 and BW % 16 == 0

    seqT_flat = sequence.T.reshape(S * B).astype(jnp.int32)
    tab_flat = table.reshape(V * D)

    mesh = plsc.VectorSubcoreMesh(core_axis_name="c", subcore_axis_name="s")

    @functools.partial(
        pl.kernel,
        out_type=jax.ShapeDtypeStruct((S, D, B), jnp.float32),
        mesh=mesh,
        scratch_types=[
            pltpu.VMEM((V * D,), jnp.float32),
            pltpu.VMEM((BW,), jnp.int32),
            pltpu.VMEM((BW,), jnp.int32),
            pltpu.VMEM((D, BW), jnp.float32),
            pltpu.VMEM((D, BW), jnp.float32),
            pltpu.SemaphoreType.DMA,
            pltpu.SemaphoreType.DMA,
            pltpu.SemaphoreType.DMA,
            pltpu.SemaphoreType.DMA,
        ],
        compiler_params=pltpu.CompilerParams(needs_layout_passes=False),
    )
    def emb(seq_hbm, tab_hbm, out_hbm,
            tab_v, idx0_v, idx1_v, rows0_v, rows1_v,
            isem0, isem1, ssem0, ssem1):
        cid = lax.axis_index("c")
        sid = lax.axis_index("s")
        wid = sid * NC + cid
        b0 = wid * BW

        idx_refs = (idx0_v, idx1_v)
        rows_refs = (rows0_v, rows1_v)
        isems = (isem0, isem1)
        ssems = (ssem0, ssem1)

        pltpu.sync_copy(tab_hbm, tab_v)

        def idx_copy(s, p):
            return pltpu.make_async_copy(
                seq_hbm.at[pl.ds(s * B + b0, BW)], idx_refs[p], isems[p]
            )

        def store_copy(s, p):
            return pltpu.make_async_copy(
                rows_refs[p], out_hbm.at[s, :, pl.ds(b0, BW)], ssems[p]
            )

        iota16 = lax.iota(jnp.int32, 16)
        rots = [(iota16 + r) % 16 for r in range(16)]

        def compute(idx_ref, rows_ref):
            @pl.loop(0, BW // 16)
            def jblock(j):
                vi = idx_ref[pl.ds(j * 16, 16)]
                vi64 = vi * D
                colv = iota16 + j * 16
                for d0 in range(0, D, 16):
                    for r in range(16):
                        ridx = rots[r] + d0
                        g = plsc.load_gather(tab_v, [vi64 + ridx])
                        plsc.store_scatter(rows_ref, [ridx, colv], g)

        idx_copy(0, 0).start()

        @pl.loop(0, S // 2)
        def jloop(j):
            for p in range(2):
                s = 2 * j + p
                idx_copy(s, p).wait()
                if p == 0:
                    idx_copy(s + 1, 1 - p).start()
                else:
                    @pl.when(j < S // 2 - 1)
                    def _prefetch(s=s, p=p):
                        idx_copy(s + 1, 1 - p).start()

                @pl.when(j >= 1)
                def _wait_store(s=s, p=p):
                    store_copy(s - 2, p).wait()

                compute(idx_refs[p], rows_refs[p])
                store_copy(s, p).start()

        store_copy(S - 2, 0).wait()
        store_copy(S - 1, 1).wait()

    out = emb(seqT_flat, tab_flat)
    return out.transpose(2, 0, 1)


# speedup vs baseline: 2.1391x; 2.1391x over previous
import functools

import jax
import jax.numpy as jnp
from jax import lax
from jax.experimental import pallas as pl
from jax.experimental.pallas import tpu as pltpu
from jax.experimental.pallas import tpu_sc as plsc

NC, NS = 2, 16
NW = NC * NS


def kernel(sequence, table):
    B, S = sequence.shape
    V, D = table.shape
    BW = B // NW
    assert BW * NW == B and S % 2 == 0 and BW % 16 == 0

    seqT_flat = sequence.T.reshape(S * B).astype(jnp.int32)
    tab_flat = table.reshape(V * D)

    mesh = plsc.VectorSubcoreMesh(core_axis_name="c", subcore_axis_name="s")

    @functools.partial(
        pl.kernel,
        out_type=jax.ShapeDtypeStruct((S, D, B), jnp.float32),
        mesh=mesh,
        scratch_types=[
            pltpu.VMEM((V * D,), jnp.float32),
            pltpu.VMEM((BW,), jnp.int32),
            pltpu.VMEM((BW,), jnp.int32),
            pltpu.VMEM((D, BW), jnp.float32),
            pltpu.VMEM((D, BW), jnp.float32),
            pltpu.SemaphoreType.DMA,
            pltpu.SemaphoreType.DMA,
            pltpu.SemaphoreType.DMA,
            pltpu.SemaphoreType.DMA,
        ],
        compiler_params=pltpu.CompilerParams(needs_layout_passes=False),
    )
    def emb(seq_hbm, tab_hbm, out_hbm,
            tab_v, idx0_v, idx1_v, rows0_v, rows1_v,
            isem0, isem1, ssem0, ssem1):
        cid = lax.axis_index("c")
        sid = lax.axis_index("s")
        wid = sid * NC + cid
        b0 = wid * BW

        idx_refs = (idx0_v, idx1_v)
        rows_refs = (rows0_v, rows1_v)
        isems = (isem0, isem1)
        ssems = (ssem0, ssem1)

        pltpu.sync_copy(tab_hbm, tab_v)

        def idx_copy(s, p):
            return pltpu.make_async_copy(
                seq_hbm.at[pl.ds(s * B + b0, BW)], idx_refs[p], isems[p]
            )

        def store_copy(s, p):
            return pltpu.make_async_copy(
                rows_refs[p], out_hbm.at[s, :, pl.ds(b0, BW)], ssems[p]
            )

        iota16 = lax.iota(jnp.int32, 16)
        iotas = [iota16 + d0 for d0 in range(0, D, 16)]

        def compute(idx_ref, rows_ref):
            @pl.loop(0, BW // 16)
            def jblock(j):
                vi = idx_ref[pl.ds(j * 16, 16)]
                vi64 = vi * D
                colv = iota16 + j * 16
                for q in range(D // 16):
                    for rb in range(0, 16, 4):
                        pairs = []
                        for r in range(rb, rb + 4):
                            ridx = iotas[q] ^ r
                            g = plsc.load_gather(tab_v, [vi64 + ridx])
                            pairs.append((ridx, g))
                        for ridx, g in pairs:
                            plsc.store_scatter(rows_ref, [ridx, colv], g)

        idx_copy(0, 0).start()

        @pl.loop(0, S // 2)
        def jloop(j):
            for p in range(2):
                s = 2 * j + p
                idx_copy(s, p).wait()
                if p == 0:
                    idx_copy(s + 1, 1 - p).start()
                else:
                    @pl.when(j < S // 2 - 1)
                    def _prefetch(s=s, p=p):
                        idx_copy(s + 1, 1 - p).start()

                @pl.when(j >= 1)
                def _wait_store(s=s, p=p):
                    store_copy(s - 2, p).wait()

                compute(idx_refs[p], rows_refs[p])
                store_copy(s, p).start()

        store_copy(S - 2, 0).wait()
        store_copy(S - 1, 1).wait()

    out = emb(seqT_flat, tab_flat)
    return out.transpose(2, 0, 1)


# speedup vs baseline: 2.4477x; 1.1443x over previous
import functools

import jax
import jax.numpy as jnp
from jax import lax
from jax.experimental import pallas as pl
from jax.experimental.pallas import tpu as pltpu
from jax.experimental.pallas import tpu_sc as plsc

NC, NS = 2, 16
NW = NC * NS


def kernel(sequence, table):
    B, S = sequence.shape
    V, D = table.shape
    BW = B // NW
    assert BW * NW == B and S % 2 == 0 and BW % 16 == 0

    seqT_flat = sequence.T.reshape(S * B).astype(jnp.int32)
    tab_flat = table.reshape(V * D)

    mesh = plsc.VectorSubcoreMesh(core_axis_name="c", subcore_axis_name="s")

    @functools.partial(
        pl.kernel,
        out_type=jax.ShapeDtypeStruct((S, D, B), jnp.float32),
        mesh=mesh,
        scratch_types=[
            pltpu.VMEM((V * D,), jnp.float32),
            pltpu.VMEM((BW,), jnp.int32),
            pltpu.VMEM((BW,), jnp.int32),
            pltpu.VMEM((D, BW), jnp.float32),
            pltpu.VMEM((D, BW), jnp.float32),
            pltpu.SemaphoreType.DMA,
            pltpu.SemaphoreType.DMA,
            pltpu.SemaphoreType.DMA,
            pltpu.SemaphoreType.DMA,
        ],
        compiler_params=pltpu.CompilerParams(needs_layout_passes=False),
    )
    def emb(seq_hbm, tab_hbm, out_hbm,
            tab_v, idx0_v, idx1_v, rows0_v, rows1_v,
            isem0, isem1, ssem0, ssem1):
        cid = lax.axis_index("c")
        sid = lax.axis_index("s")
        wid = sid * NC + cid
        b0 = wid * BW

        idx_refs = (idx0_v, idx1_v)
        rows_refs = (rows0_v, rows1_v)
        isems = (isem0, isem1)
        ssems = (ssem0, ssem1)

        pltpu.sync_copy(tab_hbm, tab_v)

        def idx_copy(s, p):
            return pltpu.make_async_copy(
                seq_hbm.at[pl.ds(s * B + b0, BW)], idx_refs[p], isems[p]
            )

        def store_copy(s, p):
            return pltpu.make_async_copy(
                rows_refs[p], out_hbm.at[s, :, pl.ds(b0, BW)], ssems[p]
            )

        iota16 = lax.iota(jnp.int32, 16)

        def compute(idx_ref, rows_ref):
            @pl.loop(0, BW // 16)
            def jblock(j):
                vi = idx_ref[pl.ds(j * 16, 16)]
                vi64 = vi * D
                colv = iota16 + j * 16
                for q in range(D // 16):
                    vbase = (vi64 + q * 16) ^ iota16
                    for rb in range(0, 16, 4):
                        pairs = []
                        for r in range(rb, rb + 4):
                            gidx = vbase ^ r
                            ridx = gidx & (D - 1)
                            g = plsc.load_gather(tab_v, [gidx])
                            pairs.append((ridx, g))
                        for ridx, g in pairs:
                            plsc.store_scatter(rows_ref, [ridx, colv], g)

        idx_copy(0, 0).start()

        @pl.loop(0, S // 2)
        def jloop(j):
            for p in range(2):
                s = 2 * j + p
                idx_copy(s, p).wait()
                if p == 0:
                    idx_copy(s + 1, 1 - p).start()
                else:
                    @pl.when(j < S // 2 - 1)
                    def _prefetch(s=s, p=p):
                        idx_copy(s + 1, 1 - p).start()

                @pl.when(j >= 1)
                def _wait_store(s=s, p=p):
                    store_copy(s - 2, p).wait()

                compute(idx_refs[p], rows_refs[p])
                store_copy(s, p).start()

        store_copy(S - 2, 0).wait()
        store_copy(S - 1, 1).wait()

    out = emb(seqT_flat, tab_flat)
    return out.transpose(2, 0, 1)


# speedup vs baseline: 2.8275x; 1.1552x over previous
import functools

import jax
import jax.numpy as jnp
from jax import lax
from jax.experimental import pallas as pl
from jax.experimental.pallas import tpu as pltpu
from jax.experimental.pallas import tpu_sc as plsc

NC, NS = 2, 16
NW = NC * NS


def kernel(sequence, table):
    B, S = sequence.shape
    V, D = table.shape
    BW = B // NW
    assert BW * NW == B and S % 2 == 0 and BW % 16 == 0

    seqT_flat = sequence.T.reshape(S * B).astype(jnp.int32)
    tab_flat = table.reshape(V * D)

    mesh = plsc.VectorSubcoreMesh(core_axis_name="c", subcore_axis_name="s")

    @functools.partial(
        pl.kernel,
        out_type=jax.ShapeDtypeStruct((S, D, B), jnp.float32),
        mesh=mesh,
        scratch_types=[
            pltpu.VMEM((V * D,), jnp.float32),
            pltpu.VMEM((BW,), jnp.int32),
            pltpu.VMEM((BW,), jnp.int32),
        ] + [
            pltpu.VMEM((D, 128), jnp.float32) for _ in range(2 * (BW // 128))
        ] + [
            pltpu.SemaphoreType.DMA,
            pltpu.SemaphoreType.DMA,
            pltpu.SemaphoreType.DMA,
            pltpu.SemaphoreType.DMA,
        ],
        compiler_params=pltpu.CompilerParams(needs_layout_passes=False),
    )
    def emb(seq_hbm, tab_hbm, out_hbm,
            tab_v, idx0_v, idx1_v, *rest):
        NWS = BW // 128
        rows_all = rest[: 2 * NWS]
        isem0, isem1, ssem0, ssem1 = rest[2 * NWS:]
        cid = lax.axis_index("c")
        sid = lax.axis_index("s")
        wid = sid * NC + cid
        b0 = wid * BW

        idx_refs = (idx0_v, idx1_v)
        rows_refs = (rows_all[:NWS], rows_all[NWS:])
        isems = (isem0, isem1)
        ssems = (ssem0, ssem1)

        pltpu.sync_copy(tab_hbm, tab_v)

        def idx_copy(s, p):
            return pltpu.make_async_copy(
                seq_hbm.at[pl.ds(s * B + b0, BW)], idx_refs[p], isems[p]
            )

        def store_copies(s, p):
            return [
                pltpu.make_async_copy(
                    rows_refs[p][w],
                    out_hbm.at[s, :, pl.ds(b0 + w * 128, 128)],
                    ssems[p],
                )
                for w in range(NWS)
            ]

        iota16 = lax.iota(jnp.int32, 16)

        def compute(idx_ref, rows_sub):
            for w in range(BW // 128):
                @pl.loop(0, 8)
                def jblock(j, w=w):
                    vi = idx_ref[pl.ds(w * 128 + j * 16, 16)]
                    vi64 = vi * D
                    colv = iota16 + j * 16
                    for q in range(D // 16):
                        vbase = (vi64 + q * 16) ^ iota16
                        for rb in range(0, 16, 4):
                            pairs = []
                            for r in range(rb, rb + 4):
                                gidx = vbase ^ r
                                ridx = gidx & (D - 1)
                                g = plsc.load_gather(tab_v, [gidx])
                                pairs.append((ridx, g))
                            for ridx, g in pairs:
                                plsc.store_scatter(rows_sub[w], [ridx, colv], g)

        idx_copy(0, 0).start()

        @pl.loop(0, S // 2)
        def jloop(j):
            for p in range(2):
                s = 2 * j + p
                idx_copy(s, p).wait()
                if p == 0:
                    idx_copy(s + 1, 1 - p).start()
                else:
                    @pl.when(j < S // 2 - 1)
                    def _prefetch(s=s, p=p):
                        idx_copy(s + 1, 1 - p).start()

                @pl.when(j >= 1)
                def _wait_store(s=s, p=p):
                    for cp in store_copies(s - 2, p):
                        cp.wait()

                compute(idx_refs[p], rows_refs[p])
                for cp in store_copies(s, p):
                    cp.start()

        for cp in store_copies(S - 2, 0):
            cp.wait()
        for cp in store_copies(S - 1, 1):
            cp.wait()

    out = emb(seqT_flat, tab_flat)
    return out.transpose(2, 0, 1)
